# baseline (speedup 1.0000x reference)
import functools

import jax
import jax.numpy as jnp
from jax.experimental import pallas as pl
from jax.experimental.pallas import tpu as pltpu
from jax.experimental.pallas import tpu_sc as plsc

NUM_KEYPOINTS = 512
B, N, D = 4, 2048, 512
TM = 256
K = NUM_KEYPOINTS
NL = N // 128


def _score_kernel(ns_ref, nt_ref, a_ref, b_ref,
                  ss_ref, st_ref, al_ref, bl_ref, bn_ref, ar_ref):
    i = pl.program_id(1)
    a_raw = a_ref[0]
    ar_ref[:, pl.ds(i * TM, TM)] = a_raw

    @pl.when(i == N // TM - 1)
    def _():
        for t in range(NL):
            al_ref[0, :, t, :] = ar_ref[:, t * 128:(t + 1) * 128]

    @pl.when(i == 0)
    def _():
        b_raw = b_ref[0]
        for t in range(NL):
            bl_ref[0, :, t, :] = b_raw[:, t * 128:(t + 1) * 128]
        bn_ref[...] = b_raw / jnp.maximum(nt_ref[0, 0, :][None, :], 1e-12)

    a = a_raw / jnp.maximum(ns_ref[0, 0, :][None, :], 1e-12)
    s = jax.lax.dot_general(a, bn_ref[...], (((0,), (0,)), ((), ())),
                            preferred_element_type=jnp.float32)
    ss_ref[0, 0, :] = jnp.max(s, axis=1)
    colmax = jnp.max(s, axis=0)

    @pl.when(i == 0)
    def _():
        st_ref[0, 0, :] = colmax

    @pl.when(i > 0)
    def _():
        st_ref[0, 0, :] = jnp.maximum(st_ref[0, 0, :], colmax)


def _scores(ns, nt, src_eb, tgt_eb):
    grid = (B, N // TM)
    return pl.pallas_call(
        _score_kernel,
        grid=grid,
        in_specs=[
            pl.BlockSpec((1, 1, TM), lambda b, i: (b, 0, i)),
            pl.BlockSpec((1, 1, N), lambda b, i: (b, 0, 0)),
            pl.BlockSpec((1, D, TM), lambda b, i: (b, 0, i)),
            pl.BlockSpec((1, D, N), lambda b, i: (b, 0, 0)),
        ],
        out_specs=[
            pl.BlockSpec((1, 1, TM), lambda b, i: (b, 0, i)),
            pl.BlockSpec((1, 1, N), lambda b, i: (b, 0, 0)),
            pl.BlockSpec((1, D, NL, 128), lambda b, i: (b, 0, 0, 0)),
            pl.BlockSpec((1, D, NL, 128), lambda b, i: (b, 0, 0, 0)),
        ],
        out_shape=[
            jax.ShapeDtypeStruct((B, 1, N), jnp.float32),
            jax.ShapeDtypeStruct((B, 1, N), jnp.float32),
            jax.ShapeDtypeStruct((B, D, NL, 128), jnp.float32),
            jax.ShapeDtypeStruct((B, D, NL, 128), jnp.float32),
        ],
        scratch_shapes=[pltpu.VMEM((D, N), jnp.float32),
                        pltpu.VMEM((D, N), jnp.float32)],
    )(ns, nt, src_eb, tgt_eb)


def _rank_kernel(s_ref, pos_ref):
    s = s_ref[0, 0, :]
    col = s[:, None]
    row = s[None, :]
    idx = jax.lax.broadcasted_iota(jnp.int32, (N, N), 0)
    jdx = jax.lax.broadcasted_iota(jnp.int32, (N, N), 1)
    cnt = jnp.logical_or(row > col,
                         jnp.logical_and(row == col, jdx < idx))
    pos_ref[0, 0, :] = jnp.sum(cnt.astype(jnp.float32), axis=1)


def _ranks(score):
    return pl.pallas_call(
        _rank_kernel,
        grid=(B,),
        in_specs=[pl.BlockSpec((1, 1, N), lambda b: (b, 0, 0))],
        out_specs=pl.BlockSpec((1, 1, N), lambda b: (b, 0, 0)),
        out_shape=jax.ShapeDtypeStruct((B, 1, N), jnp.float32),
    )(score)


_RB = 8
_GROUPS = 8


def _sc_gather_body(pos_hbm, xyz_hbm, eb_hbm, kp_hbm, ebo_hbm,
                    pos_v, idx_v, row_v, out_v, xyz_v, kp_v):
    wid = jax.lax.axis_index("c") * 16 + jax.lax.axis_index("s")
    b = wid // _GROUPS
    g = wid % _GROUPS
    d0 = g * (D // _GROUPS)

    pltpu.sync_copy(pos_hbm.at[b, 0, :], pos_v)
    iota = jax.lax.iota(jnp.int32, 16)
    for c in range(N // 16):
        pi = pos_v[pl.ds(c * 16, 16)].astype(jnp.int32)
        plsc.store_scatter(idx_v, [pi], iota + (c * 16), mask=pi < K)

    def blk_body(blk, carry):
        dbase = d0 + blk * _RB
        pltpu.sync_copy(eb_hbm.at[pl.ds((b * D + dbase) * N, _RB * N)], row_v)
        for r in range(_RB):
            for k in range(K // 16):
                iv = idx_v[pl.ds(k * 16, 16)]
                out_v[pl.ds(r * K + k * 16, 16)] = plsc.load_gather(
                    row_v, [iv + r * N])
        pltpu.sync_copy(out_v, ebo_hbm.at[b, pl.ds(dbase * K, _RB * K)])
        return carry

    jax.lax.fori_loop(0, (D // _GROUPS) // _RB, blk_body, 0)

    @pl.when(g == 0)
    def _():
        pltpu.sync_copy(xyz_hbm.at[b, :], xyz_v)
        for c in range(3 * K // 16):
            t = iota + (c * 16)
            kk = jnp.right_shift(jnp.int32(21846) * t, jnp.int32(16))
            cc = t - 3 * kk
            i1 = plsc.load_gather(idx_v, [kk])
            kp_v[pl.ds(c * 16, 16)] = plsc.load_gather(xyz_v, [3 * i1 + cc])
        pltpu.sync_copy(kp_v, kp_hbm.at[b, :])


def _gather(pos, xyz, eb_lin):
    xyz_flat = xyz.reshape(B, N * 3)
    eb_flat = eb_lin.reshape(B * D * N)
    kp_flat, ebo = pl.kernel(
        _sc_gather_body,
        mesh=plsc.VectorSubcoreMesh(core_axis_name="c", subcore_axis_name="s"),
        compiler_params=pltpu.CompilerParams(needs_layout_passes=False),
        out_type=[
            jax.ShapeDtypeStruct((B, K * 3), jnp.float32),
            jax.ShapeDtypeStruct((B, D * K), jnp.float32),
        ],
        scratch_types=[
            pltpu.VMEM((N,), jnp.float32),
            pltpu.VMEM((K,), jnp.int32),
            pltpu.VMEM((_RB * N,), jnp.float32),
            pltpu.VMEM((_RB * K,), jnp.float32),
            pltpu.VMEM((N * 3,), jnp.float32),
            pltpu.VMEM((K * 3,), jnp.float32),
        ],
    )(pos, xyz_flat, eb_flat)
    return kp_flat.reshape(B, K, 3), ebo.reshape(B, D, K)


def kernel(src, tgt, src_n, tgt_n, src_eb, tgt_eb):
    ns = jnp.linalg.norm(jnp.transpose(src_eb, (0, 2, 1)), axis=2,
                         keepdims=True)
    ns = jnp.transpose(ns, (0, 2, 1))
    nt = jnp.linalg.norm(tgt_eb, axis=1, keepdims=True)
    score_src, score_tgt, src_lin, tgt_lin = _scores(ns, nt, src_eb, tgt_eb)
    pos_src = _ranks(score_src)
    pos_tgt = _ranks(score_tgt)
    src_keypoints, src_eb_out = _gather(pos_src, src, src_lin)
    tgt_keypoints, tgt_eb_out = _gather(pos_tgt, tgt, tgt_lin)
    return (src_keypoints, tgt_keypoints, src_eb_out, tgt_eb_out)


# speedup vs baseline: 1.0595x; 1.0595x over previous
import functools

import jax
import jax.numpy as jnp
from jax.experimental import pallas as pl
from jax.experimental.pallas import tpu as pltpu
from jax.experimental.pallas import tpu_sc as plsc

NUM_KEYPOINTS = 512
B, N, D = 4, 2048, 512
TM = 256
K = NUM_KEYPOINTS
NL = N // 128


def _score_kernel(ns_ref, nt_ref, a_ref, b_ref,
                  ss_ref, st_ref, al_ref, bl_ref, bn_ref):
    i = pl.program_id(1)
    a_raw = a_ref[0]
    for t in range(TM // 128):
        al_ref[0, t] = a_raw[:, t * 128:(t + 1) * 128]

    @pl.when(i == 0)
    def _():
        b_raw = b_ref[0]
        for t in range(NL):
            bl_ref[0, t] = b_raw[:, t * 128:(t + 1) * 128]
        bn_ref[...] = b_raw / jnp.maximum(nt_ref[0, 0, :][None, :], 1e-12)

    a = a_raw / jnp.maximum(ns_ref[0, 0, :][None, :], 1e-12)
    s = jax.lax.dot_general(a, bn_ref[...], (((0,), (0,)), ((), ())),
                            preferred_element_type=jnp.float32)
    ss_ref[0, 0, :] = jnp.max(s, axis=1)
    colmax = jnp.max(s, axis=0)

    @pl.when(i == 0)
    def _():
        st_ref[0, 0, :] = colmax

    @pl.when(i > 0)
    def _():
        st_ref[0, 0, :] = jnp.maximum(st_ref[0, 0, :], colmax)


def _scores(ns, nt, src_eb, tgt_eb):
    grid = (B, N // TM)
    return pl.pallas_call(
        _score_kernel,
        grid=grid,
        in_specs=[
            pl.BlockSpec((1, 1, TM), lambda b, i: (b, 0, i)),
            pl.BlockSpec((1, 1, N), lambda b, i: (b, 0, 0)),
            pl.BlockSpec((1, D, TM), lambda b, i: (b, 0, i)),
            pl.BlockSpec((1, D, N), lambda b, i: (b, 0, 0)),
        ],
        out_specs=[
            pl.BlockSpec((1, 1, TM), lambda b, i: (b, 0, i)),
            pl.BlockSpec((1, 1, N), lambda b, i: (b, 0, 0)),
            pl.BlockSpec((1, TM // 128, D, 128), lambda b, i: (b, i, 0, 0)),
            pl.BlockSpec((1, NL, D, 128), lambda b, i: (b, 0, 0, 0)),
        ],
        out_shape=[
            jax.ShapeDtypeStruct((B, 1, N), jnp.float32),
            jax.ShapeDtypeStruct((B, 1, N), jnp.float32),
            jax.ShapeDtypeStruct((B, NL, D, 128), jnp.float32),
            jax.ShapeDtypeStruct((B, NL, D, 128), jnp.float32),
        ],
        scratch_shapes=[pltpu.VMEM((D, N), jnp.float32)],
    )(ns, nt, src_eb, tgt_eb)


def _rank_kernel(s_ref, pos_ref):
    s = s_ref[0, 0, :]
    col = s[:, None]
    row = s[None, :]
    idx = jax.lax.broadcasted_iota(jnp.int32, (N, N), 0)
    jdx = jax.lax.broadcasted_iota(jnp.int32, (N, N), 1)
    cnt = jnp.logical_or(row > col,
                         jnp.logical_and(row == col, jdx < idx))
    pos_ref[0, 0, :] = jnp.sum(cnt.astype(jnp.float32), axis=1)


def _ranks(score):
    return pl.pallas_call(
        _rank_kernel,
        grid=(B,),
        in_specs=[pl.BlockSpec((1, 1, N), lambda b: (b, 0, 0))],
        out_specs=pl.BlockSpec((1, 1, N), lambda b: (b, 0, 0)),
        out_shape=jax.ShapeDtypeStruct((B, 1, N), jnp.float32),
    )(score)


_RB = 8
_GROUPS = 8


def _sc_gather_body(pos_hbm, xyz_hbm, eb_hbm, kp_hbm, ebo_hbm,
                    pos_v, idx_v, row_v, out_v, xyz_v, kp_v, sem):
    wid = jax.lax.axis_index("c") * 16 + jax.lax.axis_index("s")
    b = wid // _GROUPS
    g = wid % _GROUPS
    d0 = g * (D // _GROUPS)

    pltpu.sync_copy(pos_hbm.at[b, 0, :], pos_v)
    iota = jax.lax.iota(jnp.int32, 16)
    for c in range(N // 16):
        pi = pos_v[pl.ds(c * 16, 16)].astype(jnp.int32)
        plsc.store_scatter(idx_v, [pi], iota + (c * 16), mask=pi < K)

    def blk_body(blk, carry):
        dbase = d0 + blk * _RB
        cps = [
            pltpu.async_copy(
                eb_hbm.at[pl.ds(((b * NL + p) * D + dbase) * 128, _RB * 128)],
                row_v.at[pl.ds(p * _RB * 128, _RB * 128)], sem)
            for p in range(NL)
        ]
        for cp in cps:
            cp.wait()
        for r in range(_RB):
            for k in range(K // 16):
                iv = idx_v[pl.ds(k * 16, 16)]
                off = ((jax.lax.shift_right_logical(iv, 7) * (_RB * 128))
                       + (r * 128) + jnp.bitwise_and(iv, 127))
                out_v[pl.ds(r * K + k * 16, 16)] = plsc.load_gather(
                    row_v, [off])
        pltpu.sync_copy(out_v, ebo_hbm.at[b, pl.ds(dbase * K, _RB * K)])
        return carry

    jax.lax.fori_loop(0, (D // _GROUPS) // _RB, blk_body, 0)

    @pl.when(g == 0)
    def _():
        pltpu.sync_copy(xyz_hbm.at[b, :], xyz_v)
        for c in range(3 * K // 16):
            t = iota + (c * 16)
            kk = jnp.right_shift(jnp.int32(21846) * t, jnp.int32(16))
            cc = t - 3 * kk
            i1 = plsc.load_gather(idx_v, [kk])
            kp_v[pl.ds(c * 16, 16)] = plsc.load_gather(xyz_v, [3 * i1 + cc])
        pltpu.sync_copy(kp_v, kp_hbm.at[b, :])


def _gather(pos, xyz, eb_lin):
    xyz_flat = xyz.reshape(B, N * 3)
    eb_flat = eb_lin.reshape(B * NL * D * 128)
    kp_flat, ebo = pl.kernel(
        _sc_gather_body,
        mesh=plsc.VectorSubcoreMesh(core_axis_name="c", subcore_axis_name="s"),
        compiler_params=pltpu.CompilerParams(needs_layout_passes=False),
        out_type=[
            jax.ShapeDtypeStruct((B, K * 3), jnp.float32),
            jax.ShapeDtypeStruct((B, D * K), jnp.float32),
        ],
        scratch_types=[
            pltpu.VMEM((N,), jnp.float32),
            pltpu.VMEM((K,), jnp.int32),
            pltpu.VMEM((_RB * N,), jnp.float32),
            pltpu.VMEM((_RB * K,), jnp.float32),
            pltpu.VMEM((N * 3,), jnp.float32),
            pltpu.VMEM((K * 3,), jnp.float32),
            pltpu.SemaphoreType.DMA,
        ],
    )(pos, xyz_flat, eb_flat)
    return kp_flat.reshape(B, K, 3), ebo.reshape(B, D, K)


def kernel(src, tgt, src_n, tgt_n, src_eb, tgt_eb):
    ns = jnp.linalg.norm(jnp.transpose(src_eb, (0, 2, 1)), axis=2,
                         keepdims=True)
    ns = jnp.transpose(ns, (0, 2, 1))
    nt = jnp.linalg.norm(tgt_eb, axis=1, keepdims=True)
    score_src, score_tgt, src_lin, tgt_lin = _scores(ns, nt, src_eb, tgt_eb)
    pos_src = _ranks(score_src)
    pos_tgt = _ranks(score_tgt)
    src_keypoints, src_eb_out = _gather(pos_src, src, src_lin)
    tgt_keypoints, tgt_eb_out = _gather(pos_tgt, tgt, tgt_lin)
    return (src_keypoints, tgt_keypoints, src_eb_out, tgt_eb_out)
